# baseline (speedup 1.0000x reference)
import jax
import jax.numpy as jnp
import numpy as np
from jax import lax
from jax.experimental import pallas as pl
from jax.experimental.pallas import tpu as pltpu
from jax.experimental.pallas import tpu_sc as plsc

N = 10000
E = 320000
IN = 128
OUT = 256
H = 32

NC, NS, L = 2, 16, 16
NW = NC * NS
NPAD = 10240
NPR = NPAD - N
CH = NPAD // NS
B = 128

EW = E // NW
EWP = 10240
BA = 64
NBLKA = EWP // BA

HALF = NPAD // 2
TRASH = 256
HALFP = HALF + TRASH
EW2 = E // NS
EWP2 = 20480
NBLK2 = EWP2 // B
CROWS = HALFP // NS

RSCALE = 1.0 / np.sqrt(float(OUT))

_f32 = jnp.float32
_i32 = jnp.int32


_RB = 512
_PRE_GRID = NPAD // _RB


def _pre_body(state_ref, pos_ref, wq_ref, bq_ref, wk_ref, bkbe_ref, we0_ref,
              x_ref, p_ref, u_ref, w_ref, tot_ref, acc_ref):
    i = pl.program_id(0)
    x = jnp.concatenate([state_ref[...], pos_ref[...]], axis=-1)
    q = jnp.dot(x, wq_ref[...], preferred_element_type=_f32) + bq_ref[...]
    p = lax.dot_general(q, wk_ref[...], (((1,), (1,)), ((), ())),
                        preferred_element_type=_f32)
    x_ref[...] = x
    p_ref[...] = p
    u_ref[...] = jnp.dot(q, bkbe_ref[...], preferred_element_type=_f32)
    w_ref[...] = jnp.dot(q, we0_ref[...], preferred_element_type=_f32)

    @pl.when(i == 0)
    def _():
        acc_ref[0] = 0.0

    acc_ref[0] += jnp.sum(x[:, 1])

    @pl.when(i == _PRE_GRID - 1)
    def _():
        tot_ref[...] = jnp.full((1, 1), acc_ref[0], _f32)


def _tc_pre(state_p, pos_p, Wq, bq2, Wk, bkbe, we0):
    return pl.pallas_call(
        _pre_body,
        grid=(_PRE_GRID,),
        in_specs=[
            pl.BlockSpec((_RB, IN - 6), lambda i: (i, 0)),
            pl.BlockSpec((_RB, 6), lambda i: (i, 0)),
            pl.BlockSpec((IN, OUT), lambda i: (0, 0)),
            pl.BlockSpec((1, OUT), lambda i: (0, 0)),
            pl.BlockSpec((IN, OUT), lambda i: (0, 0)),
            pl.BlockSpec((OUT, 1), lambda i: (0, 0)),
            pl.BlockSpec((OUT, 1), lambda i: (0, 0)),
        ],
        out_specs=[
            pl.BlockSpec((_RB, IN), lambda i: (i, 0)),
            pl.BlockSpec((_RB, IN), lambda i: (i, 0)),
            pl.BlockSpec((_RB, 1), lambda i: (i, 0)),
            pl.BlockSpec((_RB, 1), lambda i: (i, 0)),
            pl.BlockSpec((1, 1), lambda i: (0, 0)),
        ],
        out_shape=[
            jax.ShapeDtypeStruct((NPAD, IN), _f32),
            jax.ShapeDtypeStruct((NPAD, IN), _f32),
            jax.ShapeDtypeStruct((NPAD, 1), _f32),
            jax.ShapeDtypeStruct((NPAD, 1), _f32),
            jax.ShapeDtypeStruct((1, 1), _f32),
        ],
        scratch_shapes=[pltpu.SMEM((1,), _f32)],
    )(state_p, pos_p, Wq, bq2, Wk, bkbe, we0)


def _sc_logits_body(p_hbm, x_hbm, src_hbm, dst_hbm, ea_hbm, u_hbm, w_hbm,
                    ex_hbm, den_hbm,
                    src_v, dst_v, dst2_v, ea_v, ex_v, u_v, w_v, zchunk,
                    pblk, xblk, den_sh, semx, semp):
    c = lax.axis_index("c")
    s = lax.axis_index("s")
    wid = c * NS + s
    ebase = wid * EWP
    lane = lax.iota(_i32, L)

    pltpu.sync_copy(src_hbm.at[pl.ds(ebase, EWP)], src_v)
    pltpu.sync_copy(dst_hbm.at[pl.ds(ebase, EWP)], dst_v)
    pltpu.sync_copy(ea_hbm.at[pl.ds(ebase, EWP)], ea_v)
    pltpu.sync_copy(u_hbm, u_v)
    pltpu.sync_copy(w_hbm, w_v)

    @pl.loop(0, EWP // L)
    def _(j):
        e = j * L
        dst2_v[e // BA, pl.ds(e % BA, L)] = dst_v[pl.ds(e, L)]

    @pl.loop(0, CH // L)
    def _(j):
        zchunk[pl.ds(j * L, L)] = jnp.zeros((L,), _f32)

    pltpu.sync_copy(zchunk, den_sh.at[pl.ds(s * CH, CH)])
    plsc.subcore_barrier()

    pltpu.async_copy(x_hbm.at[src_v.at[pl.ds(0, BA)]], xblk.at[0], semx)
    pltpu.async_copy(p_hbm.at[dst2_v.at[0]], pblk.at[0], semp)

    @pl.loop(0, NBLKA)
    def _(b):
        par = lax.rem(b, 2)
        npar = lax.rem(b + 1, 2)
        eb = b * BA
        pltpu.make_async_copy(x_hbm.at[src_v.at[pl.ds(0, BA)]],
                              xblk.at[par], semx).wait()
        pltpu.make_async_copy(p_hbm.at[dst2_v.at[0]],
                              pblk.at[par], semp).wait()

        @pl.when(b + 1 < NBLKA)
        def _():
            pltpu.async_copy(x_hbm.at[src_v.at[pl.ds(eb + BA, BA)]],
                             xblk.at[npar], semx)
            pltpu.async_copy(p_hbm.at[dst2_v.at[b + 1]], pblk.at[npar], semp)

        @pl.loop(0, BA // L)
        def _(j):
            dotv = jnp.zeros((L,), _f32)
            for ee in range(L):
                r = j * L + ee
                acc = pblk[par, r, pl.ds(0, L)] * xblk[par, r, pl.ds(0, L)]
                for d in range(1, IN // L):
                    acc += (pblk[par, r, pl.ds(d * L, L)]
                            * xblk[par, r, pl.ds(d * L, L)])
                dotv = jnp.where(lane == ee, jnp.sum(acc), dotv)
            dstv = dst2_v[b, pl.ds(j * L, L)]
            uv = plsc.load_gather(u_v, [dstv])
            wv = plsc.load_gather(w_v, [dstv])
            lg = (dotv + uv + ea_v[pl.ds(eb + j * L, L)] * wv)
            lg = jnp.minimum(lg * RSCALE, 60.0)
            ex_v[pl.ds(eb + j * L, L)] = jnp.exp(lg)

        pltpu.sync_copy(ex_v.at[pl.ds(eb, BA)], den_sh.at[dst2_v.at[b]],
                        add=True)

    pltpu.sync_copy(ex_v, ex_hbm.at[pl.ds(ebase, EWP)])
    plsc.subcore_barrier()
    pltpu.sync_copy(den_sh.at[pl.ds(s * CH, CH)], zchunk)
    pltpu.sync_copy(zchunk, den_hbm.at[pl.ds(c * NPAD + s * CH, CH)])


def _sc_logits(P, X, srcA, dstA, eaA, u, w):
    mesh = plsc.VectorSubcoreMesh(core_axis_name="c", subcore_axis_name="s",
                                  num_cores=NC, num_subcores=NS)
    return pl.kernel(
        _sc_logits_body,
        out_type=[
            jax.ShapeDtypeStruct((NW * EWP,), _f32),
            jax.ShapeDtypeStruct((NC * NPAD,), _f32),
        ],
        mesh=mesh,
        compiler_params=pltpu.CompilerParams(needs_layout_passes=False),
        scratch_types=[
            pltpu.VMEM((EWP,), _i32),
            pltpu.VMEM((EWP,), _i32),
            pltpu.VMEM((NBLKA, BA), _i32),
            pltpu.VMEM((EWP,), _f32),
            pltpu.VMEM((EWP,), _f32),
            pltpu.VMEM((NPAD,), _f32),
            pltpu.VMEM((NPAD,), _f32),
            pltpu.VMEM((CH,), _f32),
            pltpu.VMEM((2, BA, IN), _f32),
            pltpu.VMEM((2, BA, IN), _f32),
            pltpu.VMEM_SHARED((NPAD,), _f32),
            pltpu.SemaphoreType.DMA,
            pltpu.SemaphoreType.DMA,
        ],
    )(P, X, srcA, dstA, eaA, u, w)


def _sc_agg_body(x_hbm, src_hbm, dst_hbm, ea_hbm, exg_hbm, denp_hbm,
                 yacc_hbm, sp1_hbm, sp0_hbm,
                 srcb, dstb, dst2b, eab, exb, den_v, dchunk, dsum,
                 al_v, ale_v, xblk, oblk, acc_sh, sp1_sh, sp0_sh, den_sh,
                 sem, semst):
    c = lax.axis_index("c")
    s = lax.axis_index("s")
    ebase = s * EWP2
    lane = lax.iota(_i32, L)

    pltpu.sync_copy(denp_hbm.at[pl.ds(s * CH, CH)], dchunk.at[0])
    pltpu.sync_copy(denp_hbm.at[pl.ds(NPAD + s * CH, CH)], dchunk.at[1])

    @pl.loop(0, CH // L)
    def _(j):
        dsum[pl.ds(j * L, L)] = (dchunk[0, pl.ds(j * L, L)]
                                 + dchunk[1, pl.ds(j * L, L)])

    pltpu.sync_copy(dsum, den_sh.at[pl.ds(s * CH, CH)])

    @pl.loop(0, CH // L)
    def _(j):
        dsum[pl.ds(j * L, L)] = jnp.zeros((L,), _f32)

    pltpu.sync_copy(dsum.at[pl.ds(0, CROWS)], sp1_sh.at[pl.ds(s * CROWS, CROWS)])
    pltpu.sync_copy(dsum.at[pl.ds(0, CROWS)], sp0_sh.at[pl.ds(s * CROWS, CROWS)])

    @pl.loop(0, B)
    def _(r):
        for d in range(IN // L):
            oblk[r, pl.ds(d * L, L)] = jnp.zeros((L,), _f32)

    pltpu.sync_copy(oblk, acc_sh.at[pl.ds(s * CROWS, B)])
    pltpu.sync_copy(oblk, acc_sh.at[pl.ds(s * CROWS + B, B)])
    pltpu.sync_copy(oblk.at[pl.ds(0, CROWS - 2 * B)],
                    acc_sh.at[pl.ds(s * CROWS + 2 * B, CROWS - 2 * B)])

    plsc.subcore_barrier()
    pltpu.sync_copy(den_sh, den_v)

    def _stage(b, par):
        eb = ebase + b * B
        pltpu.async_copy(src_hbm.at[pl.ds(eb, B)], srcb.at[par], semst)
        pltpu.async_copy(dst_hbm.at[pl.ds(eb, B)], dstb.at[par], semst)
        pltpu.async_copy(ea_hbm.at[pl.ds(eb, B)], eab.at[par], semst)
        pltpu.async_copy(exg_hbm.at[pl.ds(eb, B)], exb.at[par], semst)

    def _wait_stage(par):
        pltpu.make_async_copy(src_hbm.at[pl.ds(ebase, B)], srcb.at[par],
                              semst).wait()
        pltpu.make_async_copy(dst_hbm.at[pl.ds(ebase, B)], dstb.at[par],
                              semst).wait()
        pltpu.make_async_copy(ea_hbm.at[pl.ds(ebase, B)], eab.at[par],
                              semst).wait()
        pltpu.make_async_copy(exg_hbm.at[pl.ds(ebase, B)], exb.at[par],
                              semst).wait()

    _stage(0, 0)
    _stage(1, 1)
    _wait_stage(0)
    pltpu.async_copy(x_hbm.at[srcb.at[0]], xblk.at[0], sem)

    @pl.loop(0, NBLK2)
    def _(b):
        par = lax.rem(b, 2)
        npar = lax.rem(b + 1, 2)
        pltpu.make_async_copy(x_hbm.at[srcb.at[0]], xblk.at[par], sem).wait()

        @pl.when(b + 1 < NBLK2)
        def _():
            _wait_stage(npar)
            pltpu.async_copy(x_hbm.at[srcb.at[npar]], xblk.at[npar], sem)

        @pl.loop(0, B // L)
        def _(j):
            dv16 = dstb[par, pl.ds(j * L, L)]
            local = dv16 - c * HALF
            ok = (local >= 0) & (local < HALF)
            dst2b[0, pl.ds(j * L, L)] = jnp.where(ok, local,
                                                  HALF + j * L + lane)
            den16 = plsc.load_gather(den_v, [dv16])
            al16 = exb[par, pl.ds(j * L, L)] / (den16 + 1e-16)
            ea16 = eab[par, pl.ds(j * L, L)]
            al_v[pl.ds(j * L, L)] = al16
            ale_v[pl.ds(j * L, L)] = al16 * ea16
            for ee in range(L):
                r = j * L + ee
                av = jnp.broadcast_to(al16[ee], (L,))
                for d in range(IN // L):
                    oblk[r, pl.ds(d * L, L)] = xblk[par, r, pl.ds(d * L, L)] * av

        pltpu.sync_copy(oblk, acc_sh.at[dst2b.at[0]], add=True)
        pltpu.sync_copy(al_v, sp0_sh.at[dst2b.at[0]], add=True)
        pltpu.sync_copy(ale_v, sp1_sh.at[dst2b.at[0]], add=True)

        @pl.when(b + 2 < NBLK2)
        def _():
            _stage(b + 2, par)

    plsc.subcore_barrier()
    pltpu.sync_copy(acc_sh.at[pl.ds(s * CROWS, CROWS)],
                    yacc_hbm.at[c, pl.ds(s * CROWS, CROWS)])
    pltpu.sync_copy(sp1_sh.at[pl.ds(s * CROWS, CROWS)], dsum.at[pl.ds(0, CROWS)])
    pltpu.sync_copy(dsum.at[pl.ds(0, CROWS)],
                    sp1_hbm.at[pl.ds(c * HALFP + s * CROWS, CROWS)])
    pltpu.sync_copy(sp0_sh.at[pl.ds(s * CROWS, CROWS)], dsum.at[pl.ds(0, CROWS)])
    pltpu.sync_copy(dsum.at[pl.ds(0, CROWS)],
                    sp0_hbm.at[pl.ds(c * HALFP + s * CROWS, CROWS)])


def _sc_agg(X, srcC, dstC, eaC, exC, denp):
    mesh = plsc.VectorSubcoreMesh(core_axis_name="c", subcore_axis_name="s",
                                  num_cores=NC, num_subcores=NS)
    return pl.kernel(
        _sc_agg_body,
        out_type=[
            jax.ShapeDtypeStruct((NC, HALFP, IN), _f32),
            jax.ShapeDtypeStruct((NC * HALFP,), _f32),
            jax.ShapeDtypeStruct((NC * HALFP,), _f32),
        ],
        mesh=mesh,
        compiler_params=pltpu.CompilerParams(needs_layout_passes=False),
        scratch_types=[
            pltpu.VMEM((2, B), _i32),
            pltpu.VMEM((2, B), _i32),
            pltpu.VMEM((1, B), _i32),
            pltpu.VMEM((2, B), _f32),
            pltpu.VMEM((2, B), _f32),
            pltpu.VMEM((NPAD,), _f32),
            pltpu.VMEM((NC, CH), _f32),
            pltpu.VMEM((CH,), _f32),
            pltpu.VMEM((B,), _f32),
            pltpu.VMEM((B,), _f32),
            pltpu.VMEM((2, B, IN), _f32),
            pltpu.VMEM((B, IN), _f32),
            pltpu.VMEM_SHARED((HALFP, IN), _f32),
            pltpu.VMEM_SHARED((HALFP,), _f32),
            pltpu.VMEM_SHARED((HALFP,), _f32),
            pltpu.VMEM_SHARED((NPAD,), _f32),
            pltpu.SemaphoreType.DMA,
            pltpu.SemaphoreType.DMA,
        ],
    )(X, srcC, dstC, eaC, exC, denp)


_PB = 512
_POST_GRID = NPAD // _PB
_HB = HALF // _PB


def _layer_norm_leaky(h, g, b):
    m = jnp.mean(h, axis=-1, keepdims=True)
    v = jnp.mean((h - m) * (h - m), axis=-1, keepdims=True)
    h = (h - m) / jnp.sqrt(v + 1e-5) * g + b
    return jnp.where(h > 0, h, 0.01 * h)


def _post_body(yacc_ref, s1p_ref, s0p_ref, x_ref, wv_ref, we0_ref,
               bvbe_ref, wskip_ref, bskip_ref, w1a_ref, w1b_ref, w1t_ref,
               b1_ref, g1_ref, bt1_ref, w2_ref, b2_ref, g2_ref, bt2_ref,
               w3_ref, b3_ref, tot_ref, conc_ref):
    y = yacc_ref[0]
    s1 = s1p_ref[0]
    s0 = s0p_ref[0]
    x = x_ref[...]
    agg = (jnp.dot(y, wv_ref[...], preferred_element_type=_f32)
           + s1 * we0_ref[...] + s0 * bvbe_ref[...])
    out1 = agg + jnp.dot(x, wskip_ref[...], preferred_element_type=_f32)
    out1 = jnp.maximum(out1 + bskip_ref[...], 0.0)
    h = (jnp.dot(out1, w1a_ref[...], preferred_element_type=_f32)
         + jnp.dot(x, w1b_ref[...], preferred_element_type=_f32)
         + tot_ref[0, 0] * w1t_ref[...] + b1_ref[...])
    h = _layer_norm_leaky(h, g1_ref[...], bt1_ref[...])
    h = jnp.dot(h, w2_ref[...], preferred_element_type=_f32) + b2_ref[...]
    h = _layer_norm_leaky(h, g2_ref[...], bt2_ref[...])
    z = jnp.dot(h, w3_ref[...], preferred_element_type=_f32) + b3_ref[...]
    conc_ref[...] = jnp.maximum(z, 0.0) + jnp.log1p(jnp.exp(-jnp.abs(z)))


def _tc_post(yacc, s1p, s0p, X, Wv, we0r, bvbe_r, Wskip, bskip2, w1a, w1b,
             w1t, b12, g12, bt12, W2, b22, g22, bt22, W3, b32, total):
    full = lambda shp: pl.BlockSpec(shp, lambda i: tuple(0 for _ in shp))
    return pl.pallas_call(
        _post_body,
        grid=(_POST_GRID,),
        in_specs=[
            pl.BlockSpec((1, _PB, IN), lambda i: (i // _HB, i % _HB, 0)),
            pl.BlockSpec((1, _PB, 1), lambda i: (i // _HB, i % _HB, 0)),
            pl.BlockSpec((1, _PB, 1), lambda i: (i // _HB, i % _HB, 0)),
            pl.BlockSpec((_PB, IN), lambda i: (i, 0)),
            full((IN, OUT)), full((1, OUT)), full((1, OUT)),
            full((IN, OUT)), full((1, OUT)),
            full((OUT, H)), full((IN, H)), full((1, H)), full((1, H)),
            full((1, H)), full((1, H)),
            full((H, H)), full((1, H)), full((1, H)), full((1, H)),
            full((H, 1)), full((1, 1)), full((1, 1)),
        ],
        out_specs=pl.BlockSpec((_PB, 1), lambda i: (i, 0)),
        out_shape=jax.ShapeDtypeStruct((NPAD, 1), _f32),
    )(yacc, s1p, s0p, X, Wv, we0r, bvbe_r, Wskip, bskip2, w1a, w1b, w1t,
      b12, g12, bt12, W2, b22, g22, bt22, W3, b32, total)


def _norm_body(conc_ref, out_ref):
    c = conc_ref[pl.ds(0, N), :]
    out_ref[...] = c / (jnp.sum(c) + 1e-20)


def _tc_norm(conc):
    return pl.pallas_call(
        _norm_body,
        out_shape=jax.ShapeDtypeStruct((N, 1), _f32),
    )(conc)


def _pad_edges(arr, nrows, nreal, npadded, pad):
    tail = jnp.broadcast_to(pad, (nrows, npadded - nreal)).astype(arr.dtype)
    return jnp.concatenate([arr.reshape(nrows, nreal), tail], axis=1).reshape(-1)


def kernel(state, edge_index, edge_attr, pos_feat, Wq, bq, Wk, bk, Wv, bv,
           We, be, Wskip, bskip, W1, b1, g1, bt1, W2, b2, g2, bt2, W3, b3):
    state_p = jnp.pad(state, ((0, NPAD - N), (0, 0)))
    pos_p = jnp.pad(pos_feat, ((0, NPAD - N), (0, 0)))
    src = edge_index[0]
    dst = edge_index[1]
    ea = edge_attr.reshape(E)

    padA = N + jnp.arange(EWP - EW, dtype=_i32) % NPR
    padC = N + jnp.arange(EWP2 - EW2, dtype=_i32) % NPR
    srcA = _pad_edges(src, NW, EW, EWP, padA)
    dstA = _pad_edges(dst, NW, EW, EWP, padA)
    eaA = _pad_edges(ea, NW, EW, EWP, jnp.float32(0))
    srcC = _pad_edges(src, NS, EW2, EWP2, padC)
    dstC = _pad_edges(dst, NS, EW2, EWP2, padC)
    eaC = _pad_edges(ea, NS, EW2, EWP2, jnp.float32(0))

    X, P, u, w, total = _tc_pre(
        state_p, pos_p, Wq, bq.reshape(1, OUT), Wk,
        (bk + be).reshape(OUT, 1), We[0].reshape(OUT, 1))

    exA, denp = _sc_logits(P, X, srcA, dstA, eaA,
                           u.reshape(NPAD), w.reshape(NPAD))
    exC = _pad_edges(exA.reshape(NW, EWP)[:, :EW].reshape(E),
                     NS, EW2, EWP2, jnp.float32(0))
    yacc, s1p, s0p = _sc_agg(X, srcC, dstC, eaC, exC, denp)

    conc = _tc_post(
        yacc, s1p.reshape(NC, HALFP, 1), s0p.reshape(NC, HALFP, 1), X, Wv,
        We[0].reshape(1, OUT), (bv + be).reshape(1, OUT),
        Wskip, bskip.reshape(1, OUT),
        W1[:OUT], W1[OUT + 1:], W1[OUT].reshape(1, H), b1.reshape(1, H),
        g1.reshape(1, H), bt1.reshape(1, H),
        W2, b2.reshape(1, H), g2.reshape(1, H), bt2.reshape(1, H),
        W3, b3.reshape(1, 1), total)

    action = _tc_norm(conc)
    return action.reshape(1, N)


# speedup vs baseline: 1.5892x; 1.5892x over previous
import jax
import jax.numpy as jnp
import numpy as np
from jax import lax
from jax.experimental import pallas as pl
from jax.experimental.pallas import tpu as pltpu
from jax.experimental.pallas import tpu_sc as plsc

N = 10000
E = 320000
IN = 128
OUT = 256
H = 32

NC, NS, L = 2, 16, 16
NW = NC * NS
NPAD = 10240
NPR = NPAD - N
CH = NPAD // NS
B = 128

EW = E // NW
EWP = 10240
BA = 64
NBLKA = EWP // BA

HALF = NPAD // 2
TRASH = 256
HALFP = HALF + TRASH
EW2 = E // NS
EWP2 = 20480
NBLK2 = EWP2 // B
CROWS = HALFP // NS

RSCALE = 1.0 / np.sqrt(float(OUT))

_f32 = jnp.float32
_i32 = jnp.int32


_RB = 512
_PRE_GRID = NPAD // _RB


def _pre_body(state_ref, pos_ref, wq_ref, bq_ref, wk_ref, bkbe_ref, we0_ref,
              x_ref, p_ref, u_ref, w_ref, tot_ref, acc_ref):
    i = pl.program_id(0)
    x = jnp.concatenate([state_ref[...], pos_ref[...]], axis=-1)
    q = jnp.dot(x, wq_ref[...], preferred_element_type=_f32) + bq_ref[...]
    p = lax.dot_general(q, wk_ref[...], (((1,), (1,)), ((), ())),
                        preferred_element_type=_f32)
    x_ref[...] = x
    p_ref[...] = p
    u_ref[...] = jnp.dot(q, bkbe_ref[...], preferred_element_type=_f32)
    w_ref[...] = jnp.dot(q, we0_ref[...], preferred_element_type=_f32)

    @pl.when(i == 0)
    def _():
        acc_ref[0] = 0.0

    acc_ref[0] += jnp.sum(x[:, 1])

    @pl.when(i == _PRE_GRID - 1)
    def _():
        tot_ref[...] = jnp.full((1, 1), acc_ref[0], _f32)


def _tc_pre(state_p, pos_p, Wq, bq2, Wk, bkbe, we0):
    return pl.pallas_call(
        _pre_body,
        grid=(_PRE_GRID,),
        in_specs=[
            pl.BlockSpec((_RB, IN - 6), lambda i: (i, 0)),
            pl.BlockSpec((_RB, 6), lambda i: (i, 0)),
            pl.BlockSpec((IN, OUT), lambda i: (0, 0)),
            pl.BlockSpec((1, OUT), lambda i: (0, 0)),
            pl.BlockSpec((IN, OUT), lambda i: (0, 0)),
            pl.BlockSpec((OUT, 1), lambda i: (0, 0)),
            pl.BlockSpec((OUT, 1), lambda i: (0, 0)),
        ],
        out_specs=[
            pl.BlockSpec((_RB, IN), lambda i: (i, 0)),
            pl.BlockSpec((_RB, IN), lambda i: (i, 0)),
            pl.BlockSpec((_RB, 1), lambda i: (i, 0)),
            pl.BlockSpec((_RB, 1), lambda i: (i, 0)),
            pl.BlockSpec((1, 1), lambda i: (0, 0)),
        ],
        out_shape=[
            jax.ShapeDtypeStruct((NPAD, IN), _f32),
            jax.ShapeDtypeStruct((NPAD, IN), _f32),
            jax.ShapeDtypeStruct((NPAD, 1), _f32),
            jax.ShapeDtypeStruct((NPAD, 1), _f32),
            jax.ShapeDtypeStruct((1, 1), _f32),
        ],
        scratch_shapes=[pltpu.SMEM((1,), _f32)],
    )(state_p, pos_p, Wq, bq2, Wk, bkbe, we0)


def _sc_logits_body(p_hbm, x_hbm, src_hbm, dst_hbm, ea_hbm, u_hbm, w_hbm,
                    ex_hbm, den_hbm,
                    src_v, dst_v, dst2_v, ea_v, ex_v, u_v, w_v, zchunk,
                    pblk, xblk, den_sh, semx, semp):
    c = lax.axis_index("c")
    s = lax.axis_index("s")
    wid = c * NS + s
    ebase = wid * EWP
    lane = lax.iota(_i32, L)

    pltpu.sync_copy(src_hbm.at[pl.ds(ebase, EWP)], src_v)
    pltpu.sync_copy(dst_hbm.at[pl.ds(ebase, EWP)], dst_v)
    pltpu.sync_copy(ea_hbm.at[pl.ds(ebase, EWP)], ea_v)
    pltpu.sync_copy(u_hbm, u_v)
    pltpu.sync_copy(w_hbm, w_v)

    @pl.loop(0, EWP // L)
    def _(j):
        e = j * L
        dst2_v[e // BA, pl.ds(e % BA, L)] = dst_v[pl.ds(e, L)]

    @pl.loop(0, CH // L)
    def _(j):
        zchunk[pl.ds(j * L, L)] = jnp.zeros((L,), _f32)

    pltpu.sync_copy(zchunk, den_sh.at[pl.ds(s * CH, CH)])
    plsc.subcore_barrier()

    pltpu.async_copy(x_hbm.at[src_v.at[pl.ds(0, BA)]], xblk.at[0], semx)
    pltpu.async_copy(p_hbm.at[dst2_v.at[0]], pblk.at[0], semp)

    @pl.loop(0, NBLKA)
    def _(b):
        par = lax.rem(b, 2)
        npar = lax.rem(b + 1, 2)
        eb = b * BA
        pltpu.make_async_copy(x_hbm.at[src_v.at[pl.ds(0, BA)]],
                              xblk.at[par], semx).wait()
        pltpu.make_async_copy(p_hbm.at[dst2_v.at[0]],
                              pblk.at[par], semp).wait()

        @pl.when(b + 1 < NBLKA)
        def _():
            pltpu.async_copy(x_hbm.at[src_v.at[pl.ds(eb + BA, BA)]],
                             xblk.at[npar], semx)
            pltpu.async_copy(p_hbm.at[dst2_v.at[b + 1]], pblk.at[npar], semp)

        def _dots(pr):
            @pl.loop(0, BA // L)
            def _(j):
                dotv = jnp.zeros((L,), _f32)
                for ee in range(L):
                    r = j * L + ee
                    acc = pblk[pr, r, pl.ds(0, L)] * xblk[pr, r, pl.ds(0, L)]
                    for d in range(1, IN // L):
                        acc += (pblk[pr, r, pl.ds(d * L, L)]
                                * xblk[pr, r, pl.ds(d * L, L)])
                    dotv = jnp.where(lane == ee, jnp.sum(acc), dotv)
                dstv = dst2_v[b, pl.ds(j * L, L)]
                uv = plsc.load_gather(u_v, [dstv])
                wv = plsc.load_gather(w_v, [dstv])
                lg = (dotv + uv + ea_v[pl.ds(eb + j * L, L)] * wv)
                lg = jnp.minimum(lg * RSCALE, 60.0)
                ex_v[pl.ds(eb + j * L, L)] = jnp.exp(lg)

        @pl.when(par == 0)
        def _():
            _dots(0)

        @pl.when(par == 1)
        def _():
            _dots(1)

        pltpu.sync_copy(ex_v.at[pl.ds(eb, BA)], den_sh.at[dst2_v.at[b]],
                        add=True)

    pltpu.sync_copy(ex_v, ex_hbm.at[pl.ds(ebase, EWP)])
    plsc.subcore_barrier()
    pltpu.sync_copy(den_sh.at[pl.ds(s * CH, CH)], zchunk)
    pltpu.sync_copy(zchunk, den_hbm.at[pl.ds(c * NPAD + s * CH, CH)])


def _sc_logits(P, X, srcA, dstA, eaA, u, w):
    mesh = plsc.VectorSubcoreMesh(core_axis_name="c", subcore_axis_name="s",
                                  num_cores=NC, num_subcores=NS)
    return pl.kernel(
        _sc_logits_body,
        out_type=[
            jax.ShapeDtypeStruct((NW * EWP,), _f32),
            jax.ShapeDtypeStruct((NC * NPAD,), _f32),
        ],
        mesh=mesh,
        compiler_params=pltpu.CompilerParams(needs_layout_passes=False),
        scratch_types=[
            pltpu.VMEM((EWP,), _i32),
            pltpu.VMEM((EWP,), _i32),
            pltpu.VMEM((NBLKA, BA), _i32),
            pltpu.VMEM((EWP,), _f32),
            pltpu.VMEM((EWP,), _f32),
            pltpu.VMEM((NPAD,), _f32),
            pltpu.VMEM((NPAD,), _f32),
            pltpu.VMEM((CH,), _f32),
            pltpu.VMEM((2, BA, IN), _f32),
            pltpu.VMEM((2, BA, IN), _f32),
            pltpu.VMEM_SHARED((NPAD,), _f32),
            pltpu.SemaphoreType.DMA,
            pltpu.SemaphoreType.DMA,
        ],
    )(P, X, srcA, dstA, eaA, u, w)


def _sc_agg_body(x_hbm, src_hbm, dst_hbm, ea_hbm, exg_hbm, denp_hbm,
                 yacc_hbm, sp1_hbm, sp0_hbm,
                 srcb, dstb, dst2b, eab, exb, den_v, dchunk, dsum,
                 al_v, ale_v, xblk, oblk, acc_sh, sp1_sh, sp0_sh, den_sh,
                 sem, semst):
    c = lax.axis_index("c")
    s = lax.axis_index("s")
    ebase = s * EWP2
    lane = lax.iota(_i32, L)

    pltpu.sync_copy(denp_hbm.at[pl.ds(s * CH, CH)], dchunk.at[0])
    pltpu.sync_copy(denp_hbm.at[pl.ds(NPAD + s * CH, CH)], dchunk.at[1])

    @pl.loop(0, CH // L)
    def _(j):
        dsum[pl.ds(j * L, L)] = (dchunk[0, pl.ds(j * L, L)]
                                 + dchunk[1, pl.ds(j * L, L)])

    pltpu.sync_copy(dsum, den_sh.at[pl.ds(s * CH, CH)])

    @pl.loop(0, CH // L)
    def _(j):
        dsum[pl.ds(j * L, L)] = jnp.zeros((L,), _f32)

    pltpu.sync_copy(dsum.at[pl.ds(0, CROWS)], sp1_sh.at[pl.ds(s * CROWS, CROWS)])
    pltpu.sync_copy(dsum.at[pl.ds(0, CROWS)], sp0_sh.at[pl.ds(s * CROWS, CROWS)])

    @pl.loop(0, B)
    def _(r):
        for d in range(IN // L):
            oblk[r, pl.ds(d * L, L)] = jnp.zeros((L,), _f32)

    pltpu.sync_copy(oblk, acc_sh.at[pl.ds(s * CROWS, B)])
    pltpu.sync_copy(oblk, acc_sh.at[pl.ds(s * CROWS + B, B)])
    pltpu.sync_copy(oblk.at[pl.ds(0, CROWS - 2 * B)],
                    acc_sh.at[pl.ds(s * CROWS + 2 * B, CROWS - 2 * B)])

    plsc.subcore_barrier()
    pltpu.sync_copy(den_sh, den_v)

    def _stage(b, par):
        eb = ebase + b * B
        pltpu.async_copy(src_hbm.at[pl.ds(eb, B)], srcb.at[par], semst)
        pltpu.async_copy(dst_hbm.at[pl.ds(eb, B)], dstb.at[par], semst)
        pltpu.async_copy(ea_hbm.at[pl.ds(eb, B)], eab.at[par], semst)
        pltpu.async_copy(exg_hbm.at[pl.ds(eb, B)], exb.at[par], semst)

    def _wait_stage(par):
        pltpu.make_async_copy(src_hbm.at[pl.ds(ebase, B)], srcb.at[par],
                              semst).wait()
        pltpu.make_async_copy(dst_hbm.at[pl.ds(ebase, B)], dstb.at[par],
                              semst).wait()
        pltpu.make_async_copy(ea_hbm.at[pl.ds(ebase, B)], eab.at[par],
                              semst).wait()
        pltpu.make_async_copy(exg_hbm.at[pl.ds(ebase, B)], exb.at[par],
                              semst).wait()

    _stage(0, 0)
    _stage(1, 1)
    _wait_stage(0)
    pltpu.async_copy(x_hbm.at[srcb.at[0]], xblk.at[0], sem)

    @pl.loop(0, NBLK2)
    def _(b):
        par = lax.rem(b, 2)
        npar = lax.rem(b + 1, 2)
        pltpu.make_async_copy(x_hbm.at[srcb.at[0]], xblk.at[par], sem).wait()

        @pl.when(b + 1 < NBLK2)
        def _():
            _wait_stage(npar)
            pltpu.async_copy(x_hbm.at[srcb.at[npar]], xblk.at[npar], sem)

        def _wrows(pr):
            @pl.loop(0, B // L)
            def _(j):
                dv16 = dstb[pr, pl.ds(j * L, L)]
                local = dv16 - c * HALF
                ok = (local >= 0) & (local < HALF)
                dst2b[0, pl.ds(j * L, L)] = jnp.where(ok, local,
                                                      HALF + j * L + lane)
                den16 = plsc.load_gather(den_v, [dv16])
                al16 = exb[pr, pl.ds(j * L, L)] / (den16 + 1e-16)
                ea16 = eab[pr, pl.ds(j * L, L)]
                al_v[pl.ds(j * L, L)] = al16
                ale_v[pl.ds(j * L, L)] = al16 * ea16
                for ee in range(L):
                    r = j * L + ee
                    av = jnp.broadcast_to(al16[ee], (L,))
                    for d in range(IN // L):
                        oblk[r, pl.ds(d * L, L)] = xblk[pr, r, pl.ds(d * L, L)] * av

        @pl.when(par == 0)
        def _():
            _wrows(0)

        @pl.when(par == 1)
        def _():
            _wrows(1)

        pltpu.sync_copy(oblk, acc_sh.at[dst2b.at[0]], add=True)
        pltpu.sync_copy(al_v, sp0_sh.at[dst2b.at[0]], add=True)
        pltpu.sync_copy(ale_v, sp1_sh.at[dst2b.at[0]], add=True)

        @pl.when(b + 2 < NBLK2)
        def _():
            _stage(b + 2, par)

    plsc.subcore_barrier()
    pltpu.sync_copy(acc_sh.at[pl.ds(s * CROWS, CROWS)],
                    yacc_hbm.at[c, pl.ds(s * CROWS, CROWS)])
    pltpu.sync_copy(sp1_sh.at[pl.ds(s * CROWS, CROWS)], dsum.at[pl.ds(0, CROWS)])
    pltpu.sync_copy(dsum.at[pl.ds(0, CROWS)],
                    sp1_hbm.at[pl.ds(c * HALFP + s * CROWS, CROWS)])
    pltpu.sync_copy(sp0_sh.at[pl.ds(s * CROWS, CROWS)], dsum.at[pl.ds(0, CROWS)])
    pltpu.sync_copy(dsum.at[pl.ds(0, CROWS)],
                    sp0_hbm.at[pl.ds(c * HALFP + s * CROWS, CROWS)])


def _sc_agg(X, srcC, dstC, eaC, exC, denp):
    mesh = plsc.VectorSubcoreMesh(core_axis_name="c", subcore_axis_name="s",
                                  num_cores=NC, num_subcores=NS)
    return pl.kernel(
        _sc_agg_body,
        out_type=[
            jax.ShapeDtypeStruct((NC, HALFP, IN), _f32),
            jax.ShapeDtypeStruct((NC * HALFP,), _f32),
            jax.ShapeDtypeStruct((NC * HALFP,), _f32),
        ],
        mesh=mesh,
        compiler_params=pltpu.CompilerParams(needs_layout_passes=False),
        scratch_types=[
            pltpu.VMEM((2, B), _i32),
            pltpu.VMEM((2, B), _i32),
            pltpu.VMEM((1, B), _i32),
            pltpu.VMEM((2, B), _f32),
            pltpu.VMEM((2, B), _f32),
            pltpu.VMEM((NPAD,), _f32),
            pltpu.VMEM((NC, CH), _f32),
            pltpu.VMEM((CH,), _f32),
            pltpu.VMEM((B,), _f32),
            pltpu.VMEM((B,), _f32),
            pltpu.VMEM((2, B, IN), _f32),
            pltpu.VMEM((B, IN), _f32),
            pltpu.VMEM_SHARED((HALFP, IN), _f32),
            pltpu.VMEM_SHARED((HALFP,), _f32),
            pltpu.VMEM_SHARED((HALFP,), _f32),
            pltpu.VMEM_SHARED((NPAD,), _f32),
            pltpu.SemaphoreType.DMA,
            pltpu.SemaphoreType.DMA,
        ],
    )(X, srcC, dstC, eaC, exC, denp)


_PB = 512
_POST_GRID = NPAD // _PB
_HB = HALF // _PB


def _layer_norm_leaky(h, g, b):
    m = jnp.mean(h, axis=-1, keepdims=True)
    v = jnp.mean((h - m) * (h - m), axis=-1, keepdims=True)
    h = (h - m) / jnp.sqrt(v + 1e-5) * g + b
    return jnp.where(h > 0, h, 0.01 * h)


def _post_body(yacc_ref, s1p_ref, s0p_ref, x_ref, wv_ref, we0_ref,
               bvbe_ref, wskip_ref, bskip_ref, w1a_ref, w1b_ref, w1t_ref,
               b1_ref, g1_ref, bt1_ref, w2_ref, b2_ref, g2_ref, bt2_ref,
               w3_ref, b3_ref, tot_ref, conc_ref):
    y = yacc_ref[0]
    s1 = s1p_ref[0]
    s0 = s0p_ref[0]
    x = x_ref[...]
    agg = (jnp.dot(y, wv_ref[...], preferred_element_type=_f32)
           + s1 * we0_ref[...] + s0 * bvbe_ref[...])
    out1 = agg + jnp.dot(x, wskip_ref[...], preferred_element_type=_f32)
    out1 = jnp.maximum(out1 + bskip_ref[...], 0.0)
    h = (jnp.dot(out1, w1a_ref[...], preferred_element_type=_f32)
         + jnp.dot(x, w1b_ref[...], preferred_element_type=_f32)
         + tot_ref[0, 0] * w1t_ref[...] + b1_ref[...])
    h = _layer_norm_leaky(h, g1_ref[...], bt1_ref[...])
    h = jnp.dot(h, w2_ref[...], preferred_element_type=_f32) + b2_ref[...]
    h = _layer_norm_leaky(h, g2_ref[...], bt2_ref[...])
    z = jnp.dot(h, w3_ref[...], preferred_element_type=_f32) + b3_ref[...]
    conc_ref[...] = jnp.maximum(z, 0.0) + jnp.log1p(jnp.exp(-jnp.abs(z)))


def _tc_post(yacc, s1p, s0p, X, Wv, we0r, bvbe_r, Wskip, bskip2, w1a, w1b,
             w1t, b12, g12, bt12, W2, b22, g22, bt22, W3, b32, total):
    full = lambda shp: pl.BlockSpec(shp, lambda i: tuple(0 for _ in shp))
    return pl.pallas_call(
        _post_body,
        grid=(_POST_GRID,),
        in_specs=[
            pl.BlockSpec((1, _PB, IN), lambda i: (i // _HB, i % _HB, 0)),
            pl.BlockSpec((1, _PB, 1), lambda i: (i // _HB, i % _HB, 0)),
            pl.BlockSpec((1, _PB, 1), lambda i: (i // _HB, i % _HB, 0)),
            pl.BlockSpec((_PB, IN), lambda i: (i, 0)),
            full((IN, OUT)), full((1, OUT)), full((1, OUT)),
            full((IN, OUT)), full((1, OUT)),
            full((OUT, H)), full((IN, H)), full((1, H)), full((1, H)),
            full((1, H)), full((1, H)),
            full((H, H)), full((1, H)), full((1, H)), full((1, H)),
            full((H, 1)), full((1, 1)), full((1, 1)),
        ],
        out_specs=pl.BlockSpec((_PB, 1), lambda i: (i, 0)),
        out_shape=jax.ShapeDtypeStruct((NPAD, 1), _f32),
    )(yacc, s1p, s0p, X, Wv, we0r, bvbe_r, Wskip, bskip2, w1a, w1b, w1t,
      b12, g12, bt12, W2, b22, g22, bt22, W3, b32, total)


def _norm_body(conc_ref, out_ref):
    c = conc_ref[pl.ds(0, N), :]
    out_ref[...] = c / (jnp.sum(c) + 1e-20)


def _tc_norm(conc):
    return pl.pallas_call(
        _norm_body,
        out_shape=jax.ShapeDtypeStruct((N, 1), _f32),
    )(conc)


def _pad_edges(arr, nrows, nreal, npadded, pad):
    tail = jnp.broadcast_to(pad, (nrows, npadded - nreal)).astype(arr.dtype)
    return jnp.concatenate([arr.reshape(nrows, nreal), tail], axis=1).reshape(-1)


def kernel(state, edge_index, edge_attr, pos_feat, Wq, bq, Wk, bk, Wv, bv,
           We, be, Wskip, bskip, W1, b1, g1, bt1, W2, b2, g2, bt2, W3, b3):
    state_p = jnp.pad(state, ((0, NPAD - N), (0, 0)))
    pos_p = jnp.pad(pos_feat, ((0, NPAD - N), (0, 0)))
    src = edge_index[0]
    dst = edge_index[1]
    ea = edge_attr.reshape(E)

    padA = N + jnp.arange(EWP - EW, dtype=_i32) % NPR
    padC = N + jnp.arange(EWP2 - EW2, dtype=_i32) % NPR
    srcA = _pad_edges(src, NW, EW, EWP, padA)
    dstA = _pad_edges(dst, NW, EW, EWP, padA)
    eaA = _pad_edges(ea, NW, EW, EWP, jnp.float32(0))
    srcC = _pad_edges(src, NS, EW2, EWP2, padC)
    dstC = _pad_edges(dst, NS, EW2, EWP2, padC)
    eaC = _pad_edges(ea, NS, EW2, EWP2, jnp.float32(0))

    X, P, u, w, total = _tc_pre(
        state_p, pos_p, Wq, bq.reshape(1, OUT), Wk,
        (bk + be).reshape(OUT, 1), We[0].reshape(OUT, 1))

    exA, denp = _sc_logits(P, X, srcA, dstA, eaA,
                           u.reshape(NPAD), w.reshape(NPAD))
    exC = _pad_edges(exA.reshape(NW, EWP)[:, :EW].reshape(E),
                     NS, EW2, EWP2, jnp.float32(0))
    yacc, s1p, s0p = _sc_agg(X, srcC, dstC, eaC, exC, denp)

    conc = _tc_post(
        yacc, s1p.reshape(NC, HALFP, 1), s0p.reshape(NC, HALFP, 1), X, Wv,
        We[0].reshape(1, OUT), (bv + be).reshape(1, OUT),
        Wskip, bskip.reshape(1, OUT),
        W1[:OUT], W1[OUT + 1:], W1[OUT].reshape(1, H), b1.reshape(1, H),
        g1.reshape(1, H), bt1.reshape(1, H),
        W2, b2.reshape(1, H), g2.reshape(1, H), bt2.reshape(1, H),
        W3, b3.reshape(1, 1), total)

    action = _tc_norm(conc)
    return action.reshape(1, N)


# speedup vs baseline: 2.1165x; 1.3318x over previous
import jax
import jax.numpy as jnp
import numpy as np
from jax import lax
from jax.experimental import pallas as pl
from jax.experimental.pallas import tpu as pltpu
from jax.experimental.pallas import tpu_sc as plsc

N = 10000
E = 320000
IN = 128
OUT = 256
H = 32

NC, NS, L = 2, 16, 16
NW = NC * NS
NPAD = 10240
NPR = NPAD - N
CH = NPAD // NS
B = 128

EW = E // NW
EWP = 10240
BA = 64
NBLKA = EWP // BA

HALF = NPAD // 2
TRASH = 256
HALFP = HALF + TRASH
EW2 = E // NS
EWP2 = 20480
NBLK2 = EWP2 // B
CROWS = HALFP // NS

RSCALE = 1.0 / np.sqrt(float(OUT))

_f32 = jnp.float32
_i32 = jnp.int32


_RB = 512
_PRE_GRID = NPAD // _RB


def _pre_body(state_ref, pos_ref, wq_ref, bq_ref, wk_ref, bkbe_ref, we0_ref,
              x_ref, p_ref, u_ref, w_ref, tot_ref, acc_ref):
    i = pl.program_id(0)
    x = jnp.concatenate([state_ref[...], pos_ref[...]], axis=-1)
    q = jnp.dot(x, wq_ref[...], preferred_element_type=_f32) + bq_ref[...]
    p = lax.dot_general(q, wk_ref[...], (((1,), (1,)), ((), ())),
                        preferred_element_type=_f32)
    x_ref[...] = x
    p_ref[...] = p
    u_ref[...] = jnp.dot(q, bkbe_ref[...], preferred_element_type=_f32)
    w_ref[...] = jnp.dot(q, we0_ref[...], preferred_element_type=_f32)

    @pl.when(i == 0)
    def _():
        acc_ref[0] = 0.0

    acc_ref[0] += jnp.sum(x[:, 1])

    @pl.when(i == _PRE_GRID - 1)
    def _():
        tot_ref[...] = jnp.full((1, 1), acc_ref[0], _f32)


def _tc_pre(state_p, pos_p, Wq, bq2, Wk, bkbe, we0):
    return pl.pallas_call(
        _pre_body,
        grid=(_PRE_GRID,),
        in_specs=[
            pl.BlockSpec((_RB, IN - 6), lambda i: (i, 0)),
            pl.BlockSpec((_RB, 6), lambda i: (i, 0)),
            pl.BlockSpec((IN, OUT), lambda i: (0, 0)),
            pl.BlockSpec((1, OUT), lambda i: (0, 0)),
            pl.BlockSpec((IN, OUT), lambda i: (0, 0)),
            pl.BlockSpec((OUT, 1), lambda i: (0, 0)),
            pl.BlockSpec((OUT, 1), lambda i: (0, 0)),
        ],
        out_specs=[
            pl.BlockSpec((_RB, IN), lambda i: (i, 0)),
            pl.BlockSpec((_RB, IN), lambda i: (i, 0)),
            pl.BlockSpec((_RB, 1), lambda i: (i, 0)),
            pl.BlockSpec((_RB, 1), lambda i: (i, 0)),
            pl.BlockSpec((1, 1), lambda i: (0, 0)),
        ],
        out_shape=[
            jax.ShapeDtypeStruct((NPAD, IN), _f32),
            jax.ShapeDtypeStruct((NPAD, IN), _f32),
            jax.ShapeDtypeStruct((NPAD, 1), _f32),
            jax.ShapeDtypeStruct((NPAD, 1), _f32),
            jax.ShapeDtypeStruct((1, 1), _f32),
        ],
        scratch_shapes=[pltpu.SMEM((1,), _f32)],
    )(state_p, pos_p, Wq, bq2, Wk, bkbe, we0)


def _sc_logits_body(p_hbm, x_hbm, src_hbm, dst_hbm, ea_hbm, u_hbm, w_hbm,
                    ex_hbm, den_hbm,
                    src_v, dst_v, dst2_v, ea_v, ex_v, u_v, w_v, zchunk,
                    pblk, xblk, den_sh, semx, semp):
    c = lax.axis_index("c")
    s = lax.axis_index("s")
    wid = c * NS + s
    ebase = wid * EWP
    lane = lax.iota(_i32, L)

    pltpu.sync_copy(src_hbm.at[pl.ds(ebase, EWP)], src_v)
    pltpu.sync_copy(dst_hbm.at[pl.ds(ebase, EWP)], dst_v)
    pltpu.sync_copy(ea_hbm.at[pl.ds(ebase, EWP)], ea_v)
    pltpu.sync_copy(u_hbm, u_v)
    pltpu.sync_copy(w_hbm, w_v)

    @pl.loop(0, EWP // L)
    def _(j):
        e = j * L
        dst2_v[e // BA, pl.ds(e % BA, L)] = dst_v[pl.ds(e, L)]

    @pl.loop(0, CH // L)
    def _(j):
        zchunk[pl.ds(j * L, L)] = jnp.zeros((L,), _f32)

    pltpu.sync_copy(zchunk, den_sh.at[pl.ds(s * CH, CH)])
    plsc.subcore_barrier()

    pltpu.async_copy(x_hbm.at[src_v.at[pl.ds(0, BA)]], xblk.at[0], semx)
    pltpu.async_copy(p_hbm.at[dst2_v.at[0]], pblk.at[0], semp)

    @pl.loop(0, NBLKA)
    def _(b):
        par = lax.rem(b, 2)
        npar = lax.rem(b + 1, 2)
        eb = b * BA
        pltpu.make_async_copy(x_hbm.at[src_v.at[pl.ds(0, BA)]],
                              xblk.at[par], semx).wait()
        pltpu.make_async_copy(p_hbm.at[dst2_v.at[0]],
                              pblk.at[par], semp).wait()

        @pl.when(b + 1 < NBLKA)
        def _():
            pltpu.async_copy(x_hbm.at[src_v.at[pl.ds(eb + BA, BA)]],
                             xblk.at[npar], semx)
            pltpu.async_copy(p_hbm.at[dst2_v.at[b + 1]], pblk.at[npar], semp)

        def _dots(pr):
            @pl.loop(0, BA // L, unroll=2)
            def _(j):
                dotv = jnp.zeros((L,), _f32)
                for ee in range(L):
                    r = j * L + ee
                    acc = pblk[pr, r, pl.ds(0, L)] * xblk[pr, r, pl.ds(0, L)]
                    for d in range(1, IN // L):
                        acc += (pblk[pr, r, pl.ds(d * L, L)]
                                * xblk[pr, r, pl.ds(d * L, L)])
                    dotv = jnp.where(lane == ee, jnp.sum(acc), dotv)
                dstv = dst2_v[b, pl.ds(j * L, L)]
                uv = plsc.load_gather(u_v, [dstv])
                wv = plsc.load_gather(w_v, [dstv])
                lg = (dotv + uv + ea_v[pl.ds(eb + j * L, L)] * wv)
                lg = jnp.minimum(lg * RSCALE, 60.0)
                ex_v[pl.ds(eb + j * L, L)] = jnp.exp(lg)

        @pl.when(par == 0)
        def _():
            _dots(0)

        @pl.when(par == 1)
        def _():
            _dots(1)

        pltpu.sync_copy(ex_v.at[pl.ds(eb, BA)], den_sh.at[dst2_v.at[b]],
                        add=True)

    pltpu.sync_copy(ex_v, ex_hbm.at[pl.ds(ebase, EWP)])
    plsc.subcore_barrier()
    pltpu.sync_copy(den_sh.at[pl.ds(s * CH, CH)], zchunk)
    pltpu.sync_copy(zchunk, den_hbm.at[pl.ds(c * NPAD + s * CH, CH)])


def _sc_logits(P, X, srcA, dstA, eaA, u, w):
    mesh = plsc.VectorSubcoreMesh(core_axis_name="c", subcore_axis_name="s",
                                  num_cores=NC, num_subcores=NS)
    return pl.kernel(
        _sc_logits_body,
        out_type=[
            jax.ShapeDtypeStruct((NW * EWP,), _f32),
            jax.ShapeDtypeStruct((NC * NPAD,), _f32),
        ],
        mesh=mesh,
        compiler_params=pltpu.CompilerParams(needs_layout_passes=False),
        scratch_types=[
            pltpu.VMEM((EWP,), _i32),
            pltpu.VMEM((EWP,), _i32),
            pltpu.VMEM((NBLKA, BA), _i32),
            pltpu.VMEM((EWP,), _f32),
            pltpu.VMEM((EWP,), _f32),
            pltpu.VMEM((NPAD,), _f32),
            pltpu.VMEM((NPAD,), _f32),
            pltpu.VMEM((CH,), _f32),
            pltpu.VMEM((2, BA, IN), _f32),
            pltpu.VMEM((2, BA, IN), _f32),
            pltpu.VMEM_SHARED((NPAD,), _f32),
            pltpu.SemaphoreType.DMA,
            pltpu.SemaphoreType.DMA,
        ],
    )(P, X, srcA, dstA, eaA, u, w)


def _sc_agg_body(x_hbm, src_hbm, dst_hbm, ea_hbm, exg_hbm, denp_hbm,
                 yacc_hbm, sp1_hbm, sp0_hbm,
                 srcb, dstb, dst2b, eab, exb, den_v, dchunk, dsum,
                 al_v, ale_v, xblk, oblk, acc_sh, sp1_sh, sp0_sh, den_sh,
                 sem, semst, semsc):
    c = lax.axis_index("c")
    s = lax.axis_index("s")
    ebase = s * EWP2
    lane = lax.iota(_i32, L)

    pltpu.sync_copy(denp_hbm.at[pl.ds(s * CH, CH)], dchunk.at[0])
    pltpu.sync_copy(denp_hbm.at[pl.ds(NPAD + s * CH, CH)], dchunk.at[1])

    @pl.loop(0, CH // L)
    def _(j):
        dsum[pl.ds(j * L, L)] = (dchunk[0, pl.ds(j * L, L)]
                                 + dchunk[1, pl.ds(j * L, L)])

    pltpu.sync_copy(dsum, den_sh.at[pl.ds(s * CH, CH)])

    @pl.loop(0, CH // L)
    def _(j):
        dsum[pl.ds(j * L, L)] = jnp.zeros((L,), _f32)

    pltpu.sync_copy(dsum.at[pl.ds(0, CROWS)], sp1_sh.at[pl.ds(s * CROWS, CROWS)])
    pltpu.sync_copy(dsum.at[pl.ds(0, CROWS)], sp0_sh.at[pl.ds(s * CROWS, CROWS)])

    @pl.loop(0, B)
    def _(r):
        for d in range(IN // L):
            oblk[0, r, pl.ds(d * L, L)] = jnp.zeros((L,), _f32)

    pltpu.sync_copy(oblk.at[0], acc_sh.at[pl.ds(s * CROWS, B)])
    pltpu.sync_copy(oblk.at[0], acc_sh.at[pl.ds(s * CROWS + B, B)])
    pltpu.sync_copy(oblk.at[0, pl.ds(0, CROWS - 2 * B)],
                    acc_sh.at[pl.ds(s * CROWS + 2 * B, CROWS - 2 * B)])

    plsc.subcore_barrier()
    pltpu.sync_copy(den_sh, den_v)

    def _wait_scat(par):
        pltpu.make_async_copy(x_hbm.at[pl.ds(0, B)], oblk.at[par],
                              semsc).wait()
        pltpu.make_async_copy(exg_hbm.at[pl.ds(0, B)], al_v.at[par],
                              semsc).wait()
        pltpu.make_async_copy(exg_hbm.at[pl.ds(0, B)], ale_v.at[par],
                              semsc).wait()

    def _stage(b, par):
        eb = ebase + b * B
        pltpu.async_copy(src_hbm.at[pl.ds(eb, B)], srcb.at[par], semst)
        pltpu.async_copy(dst_hbm.at[pl.ds(eb, B)], dstb.at[par], semst)
        pltpu.async_copy(ea_hbm.at[pl.ds(eb, B)], eab.at[par], semst)
        pltpu.async_copy(exg_hbm.at[pl.ds(eb, B)], exb.at[par], semst)

    def _wait_stage(par):
        pltpu.make_async_copy(src_hbm.at[pl.ds(ebase, B)], srcb.at[par],
                              semst).wait()
        pltpu.make_async_copy(dst_hbm.at[pl.ds(ebase, B)], dstb.at[par],
                              semst).wait()
        pltpu.make_async_copy(ea_hbm.at[pl.ds(ebase, B)], eab.at[par],
                              semst).wait()
        pltpu.make_async_copy(exg_hbm.at[pl.ds(ebase, B)], exb.at[par],
                              semst).wait()

    _stage(0, 0)
    _stage(1, 1)
    _wait_stage(0)
    pltpu.async_copy(x_hbm.at[srcb.at[0]], xblk.at[0], sem)

    @pl.loop(0, NBLK2)
    def _(b):
        par = lax.rem(b, 2)
        npar = lax.rem(b + 1, 2)
        pltpu.make_async_copy(x_hbm.at[srcb.at[0]], xblk.at[par], sem).wait()

        @pl.when(b + 1 < NBLK2)
        def _():
            _wait_stage(npar)
            pltpu.async_copy(x_hbm.at[srcb.at[npar]], xblk.at[npar], sem)

        @pl.when(b >= 2)
        def _():
            _wait_scat(par)

        def _wrows(pr):
            @pl.loop(0, B // L)
            def _(j):
                dv16 = dstb[pr, pl.ds(j * L, L)]
                local = dv16 - c * HALF
                ok = (local >= 0) & (local < HALF)
                dst2b[pr, pl.ds(j * L, L)] = jnp.where(ok, local,
                                                       HALF + j * L + lane)
                den16 = plsc.load_gather(den_v, [dv16])
                al16 = exb[pr, pl.ds(j * L, L)] / (den16 + 1e-16)
                ea16 = eab[pr, pl.ds(j * L, L)]
                al_v[pr, pl.ds(j * L, L)] = al16
                ale_v[pr, pl.ds(j * L, L)] = al16 * ea16
                for ee in range(L):
                    r = j * L + ee
                    av = jnp.broadcast_to(al16[ee], (L,))
                    for d in range(IN // L):
                        oblk[pr, r, pl.ds(d * L, L)] = (
                            xblk[pr, r, pl.ds(d * L, L)] * av)

            pltpu.async_copy(oblk.at[pr], acc_sh.at[dst2b.at[pr]], semsc,
                             add=True)
            pltpu.async_copy(al_v.at[pr], sp0_sh.at[dst2b.at[pr]], semsc,
                             add=True)
            pltpu.async_copy(ale_v.at[pr], sp1_sh.at[dst2b.at[pr]], semsc,
                             add=True)

        @pl.when(par == 0)
        def _():
            _wrows(0)

        @pl.when(par == 1)
        def _():
            _wrows(1)

        @pl.when(b + 2 < NBLK2)
        def _():
            _stage(b + 2, par)

    _wait_scat(0)
    _wait_scat(1)
    plsc.subcore_barrier()
    pltpu.sync_copy(acc_sh.at[pl.ds(s * CROWS, CROWS)],
                    yacc_hbm.at[c, pl.ds(s * CROWS, CROWS)])
    pltpu.sync_copy(sp1_sh.at[pl.ds(s * CROWS, CROWS)], dsum.at[pl.ds(0, CROWS)])
    pltpu.sync_copy(dsum.at[pl.ds(0, CROWS)],
                    sp1_hbm.at[pl.ds(c * HALFP + s * CROWS, CROWS)])
    pltpu.sync_copy(sp0_sh.at[pl.ds(s * CROWS, CROWS)], dsum.at[pl.ds(0, CROWS)])
    pltpu.sync_copy(dsum.at[pl.ds(0, CROWS)],
                    sp0_hbm.at[pl.ds(c * HALFP + s * CROWS, CROWS)])


def _sc_agg(X, srcC, dstC, eaC, exC, denp):
    mesh = plsc.VectorSubcoreMesh(core_axis_name="c", subcore_axis_name="s",
                                  num_cores=NC, num_subcores=NS)
    return pl.kernel(
        _sc_agg_body,
        out_type=[
            jax.ShapeDtypeStruct((NC, HALFP, IN), _f32),
            jax.ShapeDtypeStruct((NC * HALFP,), _f32),
            jax.ShapeDtypeStruct((NC * HALFP,), _f32),
        ],
        mesh=mesh,
        compiler_params=pltpu.CompilerParams(needs_layout_passes=False),
        scratch_types=[
            pltpu.VMEM((2, B), _i32),
            pltpu.VMEM((2, B), _i32),
            pltpu.VMEM((2, B), _i32),
            pltpu.VMEM((2, B), _f32),
            pltpu.VMEM((2, B), _f32),
            pltpu.VMEM((NPAD,), _f32),
            pltpu.VMEM((NC, CH), _f32),
            pltpu.VMEM((CH,), _f32),
            pltpu.VMEM((2, B), _f32),
            pltpu.VMEM((2, B), _f32),
            pltpu.VMEM((2, B, IN), _f32),
            pltpu.VMEM((2, B, IN), _f32),
            pltpu.VMEM_SHARED((HALFP, IN), _f32),
            pltpu.VMEM_SHARED((HALFP,), _f32),
            pltpu.VMEM_SHARED((HALFP,), _f32),
            pltpu.VMEM_SHARED((NPAD,), _f32),
            pltpu.SemaphoreType.DMA,
            pltpu.SemaphoreType.DMA,
            pltpu.SemaphoreType.DMA,
        ],
    )(X, srcC, dstC, eaC, exC, denp)


_PB = 512
_POST_GRID = NPAD // _PB
_HB = HALF // _PB


def _layer_norm_leaky(h, g, b):
    m = jnp.mean(h, axis=-1, keepdims=True)
    v = jnp.mean((h - m) * (h - m), axis=-1, keepdims=True)
    h = (h - m) / jnp.sqrt(v + 1e-5) * g + b
    return jnp.where(h > 0, h, 0.01 * h)


def _post_body(yacc_ref, s1p_ref, s0p_ref, x_ref, wv_ref, we0_ref,
               bvbe_ref, wskip_ref, bskip_ref, w1a_ref, w1b_ref, w1t_ref,
               b1_ref, g1_ref, bt1_ref, w2_ref, b2_ref, g2_ref, bt2_ref,
               w3_ref, b3_ref, tot_ref, conc_ref):
    y = yacc_ref[0]
    s1 = s1p_ref[0]
    s0 = s0p_ref[0]
    x = x_ref[...]
    agg = (jnp.dot(y, wv_ref[...], preferred_element_type=_f32)
           + s1 * we0_ref[...] + s0 * bvbe_ref[...])
    out1 = agg + jnp.dot(x, wskip_ref[...], preferred_element_type=_f32)
    out1 = jnp.maximum(out1 + bskip_ref[...], 0.0)
    h = (jnp.dot(out1, w1a_ref[...], preferred_element_type=_f32)
         + jnp.dot(x, w1b_ref[...], preferred_element_type=_f32)
         + tot_ref[0, 0] * w1t_ref[...] + b1_ref[...])
    h = _layer_norm_leaky(h, g1_ref[...], bt1_ref[...])
    h = jnp.dot(h, w2_ref[...], preferred_element_type=_f32) + b2_ref[...]
    h = _layer_norm_leaky(h, g2_ref[...], bt2_ref[...])
    z = jnp.dot(h, w3_ref[...], preferred_element_type=_f32) + b3_ref[...]
    conc_ref[...] = jnp.maximum(z, 0.0) + jnp.log1p(jnp.exp(-jnp.abs(z)))


def _tc_post(yacc, s1p, s0p, X, Wv, we0r, bvbe_r, Wskip, bskip2, w1a, w1b,
             w1t, b12, g12, bt12, W2, b22, g22, bt22, W3, b32, total):
    full = lambda shp: pl.BlockSpec(shp, lambda i: tuple(0 for _ in shp))
    return pl.pallas_call(
        _post_body,
        grid=(_POST_GRID,),
        in_specs=[
            pl.BlockSpec((1, _PB, IN), lambda i: (i // _HB, i % _HB, 0)),
            pl.BlockSpec((1, _PB, 1), lambda i: (i // _HB, i % _HB, 0)),
            pl.BlockSpec((1, _PB, 1), lambda i: (i // _HB, i % _HB, 0)),
            pl.BlockSpec((_PB, IN), lambda i: (i, 0)),
            full((IN, OUT)), full((1, OUT)), full((1, OUT)),
            full((IN, OUT)), full((1, OUT)),
            full((OUT, H)), full((IN, H)), full((1, H)), full((1, H)),
            full((1, H)), full((1, H)),
            full((H, H)), full((1, H)), full((1, H)), full((1, H)),
            full((H, 1)), full((1, 1)), full((1, 1)),
        ],
        out_specs=pl.BlockSpec((_PB, 1), lambda i: (i, 0)),
        out_shape=jax.ShapeDtypeStruct((NPAD, 1), _f32),
    )(yacc, s1p, s0p, X, Wv, we0r, bvbe_r, Wskip, bskip2, w1a, w1b, w1t,
      b12, g12, bt12, W2, b22, g22, bt22, W3, b32, total)


def _norm_body(conc_ref, out_ref):
    c = conc_ref[pl.ds(0, N), :]
    out_ref[...] = c / (jnp.sum(c) + 1e-20)


def _tc_norm(conc):
    return pl.pallas_call(
        _norm_body,
        out_shape=jax.ShapeDtypeStruct((N, 1), _f32),
    )(conc)


def _pad_edges(arr, nrows, nreal, npadded, pad):
    tail = jnp.broadcast_to(pad, (nrows, npadded - nreal)).astype(arr.dtype)
    return jnp.concatenate([arr.reshape(nrows, nreal), tail], axis=1).reshape(-1)


def kernel(state, edge_index, edge_attr, pos_feat, Wq, bq, Wk, bk, Wv, bv,
           We, be, Wskip, bskip, W1, b1, g1, bt1, W2, b2, g2, bt2, W3, b3):
    state_p = jnp.pad(state, ((0, NPAD - N), (0, 0)))
    pos_p = jnp.pad(pos_feat, ((0, NPAD - N), (0, 0)))
    src = edge_index[0]
    dst = edge_index[1]
    ea = edge_attr.reshape(E)

    padA = N + jnp.arange(EWP - EW, dtype=_i32) % NPR
    padC = N + jnp.arange(EWP2 - EW2, dtype=_i32) % NPR
    srcA = _pad_edges(src, NW, EW, EWP, padA)
    dstA = _pad_edges(dst, NW, EW, EWP, padA)
    eaA = _pad_edges(ea, NW, EW, EWP, jnp.float32(0))
    srcC = _pad_edges(src, NS, EW2, EWP2, padC)
    dstC = _pad_edges(dst, NS, EW2, EWP2, padC)
    eaC = _pad_edges(ea, NS, EW2, EWP2, jnp.float32(0))

    X, P, u, w, total = _tc_pre(
        state_p, pos_p, Wq, bq.reshape(1, OUT), Wk,
        (bk + be).reshape(OUT, 1), We[0].reshape(OUT, 1))

    exA, denp = _sc_logits(P, X, srcA, dstA, eaA,
                           u.reshape(NPAD), w.reshape(NPAD))
    exC = _pad_edges(exA.reshape(NW, EWP)[:, :EW].reshape(E),
                     NS, EW2, EWP2, jnp.float32(0))
    yacc, s1p, s0p = _sc_agg(X, srcC, dstC, eaC, exC, denp)

    conc = _tc_post(
        yacc, s1p.reshape(NC, HALFP, 1), s0p.reshape(NC, HALFP, 1), X, Wv,
        We[0].reshape(1, OUT), (bv + be).reshape(1, OUT),
        Wskip, bskip.reshape(1, OUT),
        W1[:OUT], W1[OUT + 1:], W1[OUT].reshape(1, H), b1.reshape(1, H),
        g1.reshape(1, H), bt1.reshape(1, H),
        W2, b2.reshape(1, H), g2.reshape(1, H), bt2.reshape(1, H),
        W3, b3.reshape(1, 1), total)

    action = _tc_norm(conc)
    return action.reshape(1, N)


# speedup vs baseline: 2.1378x; 1.0101x over previous
import jax
import jax.numpy as jnp
import numpy as np
from jax import lax
from jax.experimental import pallas as pl
from jax.experimental.pallas import tpu as pltpu
from jax.experimental.pallas import tpu_sc as plsc

N = 10000
E = 320000
IN = 128
OUT = 256
H = 32

NC, NS, L = 2, 16, 16
NW = NC * NS
NPAD = 10240
NPR = NPAD - N
CH = NPAD // NS
B = 128

EW = E // NW
EWP = 10240
BA = 64
NBLKA = EWP // BA

HALF = NPAD // 2
TRASH = 256
HALFP = HALF + TRASH
EW2 = E // NS
EWP2 = 20480
NBLK2 = EWP2 // B
CROWS = HALFP // NS

RSCALE = 1.0 / np.sqrt(float(OUT))

_f32 = jnp.float32
_i32 = jnp.int32


_RB = 512
_PRE_GRID = NPAD // _RB


def _pre_body(state_ref, pos_ref, wq_ref, bq_ref, wk_ref, bkbe_ref, we0_ref,
              x_ref, p_ref, u_ref, w_ref, tot_ref, acc_ref):
    i = pl.program_id(0)
    x = jnp.concatenate([state_ref[...], pos_ref[...]], axis=-1)
    q = jnp.dot(x, wq_ref[...], preferred_element_type=_f32) + bq_ref[...]
    p = lax.dot_general(q, wk_ref[...], (((1,), (1,)), ((), ())),
                        preferred_element_type=_f32)
    x_ref[...] = x
    p_ref[...] = p
    u_ref[...] = jnp.dot(q, bkbe_ref[...], preferred_element_type=_f32)
    w_ref[...] = jnp.dot(q, we0_ref[...], preferred_element_type=_f32)

    @pl.when(i == 0)
    def _():
        acc_ref[0] = 0.0

    acc_ref[0] += jnp.sum(x[:, 1])

    @pl.when(i == _PRE_GRID - 1)
    def _():
        tot_ref[...] = jnp.full((1, 1), acc_ref[0], _f32)


def _tc_pre(state_p, pos_p, Wq, bq2, Wk, bkbe, we0):
    return pl.pallas_call(
        _pre_body,
        grid=(_PRE_GRID,),
        in_specs=[
            pl.BlockSpec((_RB, IN - 6), lambda i: (i, 0)),
            pl.BlockSpec((_RB, 6), lambda i: (i, 0)),
            pl.BlockSpec((IN, OUT), lambda i: (0, 0)),
            pl.BlockSpec((1, OUT), lambda i: (0, 0)),
            pl.BlockSpec((IN, OUT), lambda i: (0, 0)),
            pl.BlockSpec((OUT, 1), lambda i: (0, 0)),
            pl.BlockSpec((OUT, 1), lambda i: (0, 0)),
        ],
        out_specs=[
            pl.BlockSpec((_RB, IN), lambda i: (i, 0)),
            pl.BlockSpec((_RB, IN), lambda i: (i, 0)),
            pl.BlockSpec((_RB, 1), lambda i: (i, 0)),
            pl.BlockSpec((_RB, 1), lambda i: (i, 0)),
            pl.BlockSpec((1, 1), lambda i: (0, 0)),
        ],
        out_shape=[
            jax.ShapeDtypeStruct((NPAD, IN), _f32),
            jax.ShapeDtypeStruct((NPAD, IN), _f32),
            jax.ShapeDtypeStruct((NPAD, 1), _f32),
            jax.ShapeDtypeStruct((NPAD, 1), _f32),
            jax.ShapeDtypeStruct((1, 1), _f32),
        ],
        scratch_shapes=[pltpu.SMEM((1,), _f32)],
    )(state_p, pos_p, Wq, bq2, Wk, bkbe, we0)


def _sc_logits_body(p_hbm, x_hbm, src_hbm, dst_hbm, ea_hbm, u_hbm, w_hbm,
                    ex_hbm, den_hbm,
                    src_v, dst_v, dst2_v, ea_v, ex_v, u_v, w_v, zchunk,
                    pblk, xblk, den_sh, semx, semp, semd):
    c = lax.axis_index("c")
    s = lax.axis_index("s")
    wid = c * NS + s
    ebase = wid * EWP
    lane = lax.iota(_i32, L)

    pltpu.sync_copy(src_hbm.at[pl.ds(ebase, EWP)], src_v)
    pltpu.sync_copy(dst_hbm.at[pl.ds(ebase, EWP)], dst_v)
    pltpu.sync_copy(ea_hbm.at[pl.ds(ebase, EWP)], ea_v)
    pltpu.sync_copy(u_hbm, u_v)
    pltpu.sync_copy(w_hbm, w_v)

    @pl.loop(0, EWP // L)
    def _(j):
        e = j * L
        dst2_v[e // BA, pl.ds(e % BA, L)] = dst_v[pl.ds(e, L)]

    @pl.loop(0, CH // L)
    def _(j):
        zchunk[pl.ds(j * L, L)] = jnp.zeros((L,), _f32)

    pltpu.sync_copy(zchunk, den_sh.at[pl.ds(s * CH, CH)])
    plsc.subcore_barrier()

    pltpu.async_copy(x_hbm.at[src_v.at[pl.ds(0, BA)]], xblk.at[0], semx)
    pltpu.async_copy(p_hbm.at[dst2_v.at[0]], pblk.at[0], semp)

    @pl.loop(0, NBLKA)
    def _(b):
        par = lax.rem(b, 2)
        npar = lax.rem(b + 1, 2)
        eb = b * BA
        pltpu.make_async_copy(x_hbm.at[src_v.at[pl.ds(0, BA)]],
                              xblk.at[par], semx).wait()
        pltpu.make_async_copy(p_hbm.at[dst2_v.at[0]],
                              pblk.at[par], semp).wait()

        @pl.when(b + 1 < NBLKA)
        def _():
            pltpu.async_copy(x_hbm.at[src_v.at[pl.ds(eb + BA, BA)]],
                             xblk.at[npar], semx)
            pltpu.async_copy(p_hbm.at[dst2_v.at[b + 1]], pblk.at[npar], semp)

        def _dots(pr):
            @pl.loop(0, BA // L, unroll=2)
            def _(j):
                dotv = jnp.zeros((L,), _f32)
                for ee in range(L):
                    r = j * L + ee
                    acc = pblk[pr, r, pl.ds(0, L)] * xblk[pr, r, pl.ds(0, L)]
                    for d in range(1, IN // L):
                        acc += (pblk[pr, r, pl.ds(d * L, L)]
                                * xblk[pr, r, pl.ds(d * L, L)])
                    dotv = jnp.where(lane == ee, jnp.sum(acc), dotv)
                dstv = dst2_v[b, pl.ds(j * L, L)]
                uv = plsc.load_gather(u_v, [dstv])
                wv = plsc.load_gather(w_v, [dstv])
                lg = (dotv + uv + ea_v[pl.ds(eb + j * L, L)] * wv)
                lg = jnp.minimum(lg * RSCALE, 60.0)
                ex_v[pl.ds(eb + j * L, L)] = jnp.exp(lg)

        @pl.when(par == 0)
        def _():
            _dots(0)

        @pl.when(par == 1)
        def _():
            _dots(1)

        @pl.when(b >= 2)
        def _():
            pltpu.make_async_copy(ea_hbm.at[pl.ds(0, BA)],
                                  ex_v.at[pl.ds(0, BA)], semd).wait()

        pltpu.async_copy(ex_v.at[pl.ds(eb, BA)], den_sh.at[dst2_v.at[b]],
                         semd, add=True)

    pltpu.make_async_copy(ea_hbm.at[pl.ds(0, BA)], ex_v.at[pl.ds(0, BA)],
                          semd).wait()
    pltpu.make_async_copy(ea_hbm.at[pl.ds(0, BA)], ex_v.at[pl.ds(0, BA)],
                          semd).wait()
    pltpu.sync_copy(ex_v, ex_hbm.at[pl.ds(ebase, EWP)])
    plsc.subcore_barrier()
    pltpu.sync_copy(den_sh.at[pl.ds(s * CH, CH)], zchunk)
    pltpu.sync_copy(zchunk, den_hbm.at[pl.ds(c * NPAD + s * CH, CH)])


def _sc_logits(P, X, srcA, dstA, eaA, u, w):
    mesh = plsc.VectorSubcoreMesh(core_axis_name="c", subcore_axis_name="s",
                                  num_cores=NC, num_subcores=NS)
    return pl.kernel(
        _sc_logits_body,
        out_type=[
            jax.ShapeDtypeStruct((NW * EWP,), _f32),
            jax.ShapeDtypeStruct((NC * NPAD,), _f32),
        ],
        mesh=mesh,
        compiler_params=pltpu.CompilerParams(needs_layout_passes=False),
        scratch_types=[
            pltpu.VMEM((EWP,), _i32),
            pltpu.VMEM((EWP,), _i32),
            pltpu.VMEM((NBLKA, BA), _i32),
            pltpu.VMEM((EWP,), _f32),
            pltpu.VMEM((EWP,), _f32),
            pltpu.VMEM((NPAD,), _f32),
            pltpu.VMEM((NPAD,), _f32),
            pltpu.VMEM((CH,), _f32),
            pltpu.VMEM((2, BA, IN), _f32),
            pltpu.VMEM((2, BA, IN), _f32),
            pltpu.VMEM_SHARED((NPAD,), _f32),
            pltpu.SemaphoreType.DMA,
            pltpu.SemaphoreType.DMA,
            pltpu.SemaphoreType.DMA,
        ],
    )(P, X, srcA, dstA, eaA, u, w)


def _sc_agg_body(x_hbm, src_hbm, dst_hbm, ea_hbm, exg_hbm, denp_hbm,
                 yacc_hbm, sp1_hbm, sp0_hbm,
                 srcb, dstb, dst2b, eab, exb, den_v, dchunk, dsum,
                 al_v, ale_v, xblk, oblk, acc_sh, sp1_sh, sp0_sh, den_sh,
                 sem, semst, semsc):
    c = lax.axis_index("c")
    s = lax.axis_index("s")
    ebase = s * EWP2
    lane = lax.iota(_i32, L)

    pltpu.sync_copy(denp_hbm.at[pl.ds(s * CH, CH)], dchunk.at[0])
    pltpu.sync_copy(denp_hbm.at[pl.ds(NPAD + s * CH, CH)], dchunk.at[1])

    @pl.loop(0, CH // L)
    def _(j):
        dsum[pl.ds(j * L, L)] = (dchunk[0, pl.ds(j * L, L)]
                                 + dchunk[1, pl.ds(j * L, L)])

    pltpu.sync_copy(dsum, den_sh.at[pl.ds(s * CH, CH)])

    @pl.loop(0, CH // L)
    def _(j):
        dsum[pl.ds(j * L, L)] = jnp.zeros((L,), _f32)

    pltpu.sync_copy(dsum.at[pl.ds(0, CROWS)], sp1_sh.at[pl.ds(s * CROWS, CROWS)])
    pltpu.sync_copy(dsum.at[pl.ds(0, CROWS)], sp0_sh.at[pl.ds(s * CROWS, CROWS)])

    @pl.loop(0, B)
    def _(r):
        for d in range(IN // L):
            oblk[0, r, pl.ds(d * L, L)] = jnp.zeros((L,), _f32)

    pltpu.sync_copy(oblk.at[0], acc_sh.at[pl.ds(s * CROWS, B)])
    pltpu.sync_copy(oblk.at[0], acc_sh.at[pl.ds(s * CROWS + B, B)])
    pltpu.sync_copy(oblk.at[0, pl.ds(0, CROWS - 2 * B)],
                    acc_sh.at[pl.ds(s * CROWS + 2 * B, CROWS - 2 * B)])

    plsc.subcore_barrier()
    pltpu.sync_copy(den_sh, den_v)

    def _wait_scat(par):
        pltpu.make_async_copy(x_hbm.at[pl.ds(0, B)], oblk.at[par],
                              semsc).wait()
        pltpu.make_async_copy(exg_hbm.at[pl.ds(0, B)], al_v.at[par],
                              semsc).wait()
        pltpu.make_async_copy(exg_hbm.at[pl.ds(0, B)], ale_v.at[par],
                              semsc).wait()

    def _stage(b, par):
        eb = ebase + b * B
        pltpu.async_copy(src_hbm.at[pl.ds(eb, B)], srcb.at[par], semst)
        pltpu.async_copy(dst_hbm.at[pl.ds(eb, B)], dstb.at[par], semst)
        pltpu.async_copy(ea_hbm.at[pl.ds(eb, B)], eab.at[par], semst)
        pltpu.async_copy(exg_hbm.at[pl.ds(eb, B)], exb.at[par], semst)

    def _wait_stage(par):
        pltpu.make_async_copy(src_hbm.at[pl.ds(ebase, B)], srcb.at[par],
                              semst).wait()
        pltpu.make_async_copy(dst_hbm.at[pl.ds(ebase, B)], dstb.at[par],
                              semst).wait()
        pltpu.make_async_copy(ea_hbm.at[pl.ds(ebase, B)], eab.at[par],
                              semst).wait()
        pltpu.make_async_copy(exg_hbm.at[pl.ds(ebase, B)], exb.at[par],
                              semst).wait()

    _stage(0, 0)
    _stage(1, 1)
    _wait_stage(0)
    pltpu.async_copy(x_hbm.at[srcb.at[0]], xblk.at[0], sem)

    @pl.loop(0, NBLK2)
    def _(b):
        par = lax.rem(b, 2)
        npar = lax.rem(b + 1, 2)
        pltpu.make_async_copy(x_hbm.at[srcb.at[0]], xblk.at[par], sem).wait()

        @pl.when(b + 1 < NBLK2)
        def _():
            _wait_stage(npar)
            pltpu.async_copy(x_hbm.at[srcb.at[npar]], xblk.at[npar], sem)

        @pl.when(b >= 2)
        def _():
            _wait_scat(par)

        def _wrows(pr):
            @pl.loop(0, B // L)
            def _(j):
                dv16 = dstb[pr, pl.ds(j * L, L)]
                local = dv16 - c * HALF
                ok = (local >= 0) & (local < HALF)
                dst2b[pr, pl.ds(j * L, L)] = jnp.where(ok, local,
                                                       HALF + j * L + lane)
                den16 = plsc.load_gather(den_v, [dv16])
                al16 = exb[pr, pl.ds(j * L, L)] / (den16 + 1e-16)
                ea16 = eab[pr, pl.ds(j * L, L)]
                al_v[pr, pl.ds(j * L, L)] = al16
                ale_v[pr, pl.ds(j * L, L)] = al16 * ea16
                for ee in range(L):
                    r = j * L + ee
                    av = jnp.broadcast_to(al16[ee], (L,))
                    for d in range(IN // L):
                        oblk[pr, r, pl.ds(d * L, L)] = (
                            xblk[pr, r, pl.ds(d * L, L)] * av)

            pltpu.async_copy(oblk.at[pr], acc_sh.at[dst2b.at[pr]], semsc,
                             add=True)
            pltpu.async_copy(al_v.at[pr], sp0_sh.at[dst2b.at[pr]], semsc,
                             add=True)
            pltpu.async_copy(ale_v.at[pr], sp1_sh.at[dst2b.at[pr]], semsc,
                             add=True)

        @pl.when(par == 0)
        def _():
            _wrows(0)

        @pl.when(par == 1)
        def _():
            _wrows(1)

        @pl.when(b + 2 < NBLK2)
        def _():
            _stage(b + 2, par)

    _wait_scat(0)
    _wait_scat(1)
    plsc.subcore_barrier()
    pltpu.sync_copy(acc_sh.at[pl.ds(s * CROWS, CROWS)],
                    yacc_hbm.at[c, pl.ds(s * CROWS, CROWS)])
    pltpu.sync_copy(sp1_sh.at[pl.ds(s * CROWS, CROWS)], dsum.at[pl.ds(0, CROWS)])
    pltpu.sync_copy(dsum.at[pl.ds(0, CROWS)],
                    sp1_hbm.at[pl.ds(c * HALFP + s * CROWS, CROWS)])
    pltpu.sync_copy(sp0_sh.at[pl.ds(s * CROWS, CROWS)], dsum.at[pl.ds(0, CROWS)])
    pltpu.sync_copy(dsum.at[pl.ds(0, CROWS)],
                    sp0_hbm.at[pl.ds(c * HALFP + s * CROWS, CROWS)])


def _sc_agg(X, srcC, dstC, eaC, exC, denp):
    mesh = plsc.VectorSubcoreMesh(core_axis_name="c", subcore_axis_name="s",
                                  num_cores=NC, num_subcores=NS)
    return pl.kernel(
        _sc_agg_body,
        out_type=[
            jax.ShapeDtypeStruct((NC, HALFP, IN), _f32),
            jax.ShapeDtypeStruct((NC * HALFP,), _f32),
            jax.ShapeDtypeStruct((NC * HALFP,), _f32),
        ],
        mesh=mesh,
        compiler_params=pltpu.CompilerParams(needs_layout_passes=False),
        scratch_types=[
            pltpu.VMEM((2, B), _i32),
            pltpu.VMEM((2, B), _i32),
            pltpu.VMEM((2, B), _i32),
            pltpu.VMEM((2, B), _f32),
            pltpu.VMEM((2, B), _f32),
            pltpu.VMEM((NPAD,), _f32),
            pltpu.VMEM((NC, CH), _f32),
            pltpu.VMEM((CH,), _f32),
            pltpu.VMEM((2, B), _f32),
            pltpu.VMEM((2, B), _f32),
            pltpu.VMEM((2, B, IN), _f32),
            pltpu.VMEM((2, B, IN), _f32),
            pltpu.VMEM_SHARED((HALFP, IN), _f32),
            pltpu.VMEM_SHARED((HALFP,), _f32),
            pltpu.VMEM_SHARED((HALFP,), _f32),
            pltpu.VMEM_SHARED((NPAD,), _f32),
            pltpu.SemaphoreType.DMA,
            pltpu.SemaphoreType.DMA,
            pltpu.SemaphoreType.DMA,
        ],
    )(X, srcC, dstC, eaC, exC, denp)


_PB = 512
_POST_GRID = NPAD // _PB
_HB = HALF // _PB


def _layer_norm_leaky(h, g, b):
    m = jnp.mean(h, axis=-1, keepdims=True)
    v = jnp.mean((h - m) * (h - m), axis=-1, keepdims=True)
    h = (h - m) / jnp.sqrt(v + 1e-5) * g + b
    return jnp.where(h > 0, h, 0.01 * h)


def _post_body(yacc_ref, s1p_ref, s0p_ref, x_ref, wv_ref, we0_ref,
               bvbe_ref, wskip_ref, bskip_ref, w1a_ref, w1b_ref, w1t_ref,
               b1_ref, g1_ref, bt1_ref, w2_ref, b2_ref, g2_ref, bt2_ref,
               w3_ref, b3_ref, tot_ref, conc_ref):
    y = yacc_ref[0]
    s1 = s1p_ref[0]
    s0 = s0p_ref[0]
    x = x_ref[...]
    agg = (jnp.dot(y, wv_ref[...], preferred_element_type=_f32)
           + s1 * we0_ref[...] + s0 * bvbe_ref[...])
    out1 = agg + jnp.dot(x, wskip_ref[...], preferred_element_type=_f32)
    out1 = jnp.maximum(out1 + bskip_ref[...], 0.0)
    h = (jnp.dot(out1, w1a_ref[...], preferred_element_type=_f32)
         + jnp.dot(x, w1b_ref[...], preferred_element_type=_f32)
         + tot_ref[0, 0] * w1t_ref[...] + b1_ref[...])
    h = _layer_norm_leaky(h, g1_ref[...], bt1_ref[...])
    h = jnp.dot(h, w2_ref[...], preferred_element_type=_f32) + b2_ref[...]
    h = _layer_norm_leaky(h, g2_ref[...], bt2_ref[...])
    z = jnp.dot(h, w3_ref[...], preferred_element_type=_f32) + b3_ref[...]
    conc_ref[...] = jnp.maximum(z, 0.0) + jnp.log1p(jnp.exp(-jnp.abs(z)))


def _tc_post(yacc, s1p, s0p, X, Wv, we0r, bvbe_r, Wskip, bskip2, w1a, w1b,
             w1t, b12, g12, bt12, W2, b22, g22, bt22, W3, b32, total):
    full = lambda shp: pl.BlockSpec(shp, lambda i: tuple(0 for _ in shp))
    return pl.pallas_call(
        _post_body,
        grid=(_POST_GRID,),
        in_specs=[
            pl.BlockSpec((1, _PB, IN), lambda i: (i // _HB, i % _HB, 0)),
            pl.BlockSpec((1, _PB, 1), lambda i: (i // _HB, i % _HB, 0)),
            pl.BlockSpec((1, _PB, 1), lambda i: (i // _HB, i % _HB, 0)),
            pl.BlockSpec((_PB, IN), lambda i: (i, 0)),
            full((IN, OUT)), full((1, OUT)), full((1, OUT)),
            full((IN, OUT)), full((1, OUT)),
            full((OUT, H)), full((IN, H)), full((1, H)), full((1, H)),
            full((1, H)), full((1, H)),
            full((H, H)), full((1, H)), full((1, H)), full((1, H)),
            full((H, 1)), full((1, 1)), full((1, 1)),
        ],
        out_specs=pl.BlockSpec((_PB, 1), lambda i: (i, 0)),
        out_shape=jax.ShapeDtypeStruct((NPAD, 1), _f32),
    )(yacc, s1p, s0p, X, Wv, we0r, bvbe_r, Wskip, bskip2, w1a, w1b, w1t,
      b12, g12, bt12, W2, b22, g22, bt22, W3, b32, total)


def _norm_body(conc_ref, out_ref):
    c = conc_ref[pl.ds(0, N), :]
    out_ref[...] = c / (jnp.sum(c) + 1e-20)


def _tc_norm(conc):
    return pl.pallas_call(
        _norm_body,
        out_shape=jax.ShapeDtypeStruct((N, 1), _f32),
    )(conc)


def _pad_edges(arr, nrows, nreal, npadded, pad):
    tail = jnp.broadcast_to(pad, (nrows, npadded - nreal)).astype(arr.dtype)
    return jnp.concatenate([arr.reshape(nrows, nreal), tail], axis=1).reshape(-1)


def kernel(state, edge_index, edge_attr, pos_feat, Wq, bq, Wk, bk, Wv, bv,
           We, be, Wskip, bskip, W1, b1, g1, bt1, W2, b2, g2, bt2, W3, b3):
    state_p = jnp.pad(state, ((0, NPAD - N), (0, 0)))
    pos_p = jnp.pad(pos_feat, ((0, NPAD - N), (0, 0)))
    src = edge_index[0]
    dst = edge_index[1]
    ea = edge_attr.reshape(E)

    padA = N + jnp.arange(EWP - EW, dtype=_i32) % NPR
    padC = N + jnp.arange(EWP2 - EW2, dtype=_i32) % NPR
    srcA = _pad_edges(src, NW, EW, EWP, padA)
    dstA = _pad_edges(dst, NW, EW, EWP, padA)
    eaA = _pad_edges(ea, NW, EW, EWP, jnp.float32(0))
    srcC = _pad_edges(src, NS, EW2, EWP2, padC)
    dstC = _pad_edges(dst, NS, EW2, EWP2, padC)
    eaC = _pad_edges(ea, NS, EW2, EWP2, jnp.float32(0))

    X, P, u, w, total = _tc_pre(
        state_p, pos_p, Wq, bq.reshape(1, OUT), Wk,
        (bk + be).reshape(OUT, 1), We[0].reshape(OUT, 1))

    exA, denp = _sc_logits(P, X, srcA, dstA, eaA,
                           u.reshape(NPAD), w.reshape(NPAD))
    exC = _pad_edges(exA.reshape(NW, EWP)[:, :EW].reshape(E),
                     NS, EW2, EWP2, jnp.float32(0))
    yacc, s1p, s0p = _sc_agg(X, srcC, dstC, eaC, exC, denp)

    conc = _tc_post(
        yacc, s1p.reshape(NC, HALFP, 1), s0p.reshape(NC, HALFP, 1), X, Wv,
        We[0].reshape(1, OUT), (bv + be).reshape(1, OUT),
        Wskip, bskip.reshape(1, OUT),
        W1[:OUT], W1[OUT + 1:], W1[OUT].reshape(1, H), b1.reshape(1, H),
        g1.reshape(1, H), bt1.reshape(1, H),
        W2, b2.reshape(1, H), g2.reshape(1, H), bt2.reshape(1, H),
        W3, b3.reshape(1, 1), total)

    action = _tc_norm(conc)
    return action.reshape(1, N)
